# baseline (speedup 1.0000x reference)
import functools

import jax
import jax.numpy as jnp
from jax import lax
from jax.experimental import pallas as pl
from jax.experimental.pallas import tpu as pltpu
from jax.experimental.pallas import tpu_sc as plsc

HID = 768
LANES = 16
VPR = HID // LANES
NW = 32
CHUNK = 16
EPS = 1e-5


def _allsum16(x):
    idx = lax.iota(jnp.int32, LANES)
    dnums = lax.GatherDimensionNumbers(
        offset_dims=(), collapsed_slice_dims=(0,), start_index_map=(0,))
    for k in (8, 4, 2, 1):
        x = x + lax.gather(x, (idx ^ k)[:, None], dnums, slice_sizes=(1,),
                           mode=lax.GatherScatterMode.PROMISE_IN_BOUNDS)
    return x


def _rsqrt16(x):
    i = lax.bitcast_convert_type(x, jnp.int32)
    y = lax.bitcast_convert_type(jnp.int32(0x5F3759DF) - (i >> 1), jnp.float32)
    for _ in range(3):
        y = y * (1.5 - 0.5 * x * y * y)
    return y


def _emb_body(tok_t, seg_t, pos_t, tid, sid, pid, gam, bet, out,
              idx_t, idx_s, idx_p, rt0, rp0, rt1, rp1, ro0, ro1, xT, sv,
              gv, bv, sg0, sg1, ss0, ss1, tpw, nchunk):
    wid = lax.axis_index("s") * 2 + lax.axis_index("c")
    base = wid * tpw
    pltpu.sync_copy(tid.at[pl.ds(base, tpw)], idx_t)
    pltpu.sync_copy(sid.at[pl.ds(base, tpw)], idx_s.at[pl.ds(0, tpw)])
    pltpu.sync_copy(pid.at[pl.ds(base, tpw)], idx_p)
    pltpu.sync_copy(seg_t, sv)
    pltpu.sync_copy(gam, gv)
    pltpu.sync_copy(bet, bv)

    def gstart(c, rt, rp, sem):
        off = c * CHUNK
        pltpu.async_copy(tok_t.at[idx_t.at[pl.ds(off, CHUNK)]], rt, sem)
        pltpu.async_copy(pos_t.at[idx_p.at[pl.ds(off, CHUNK)]], rp, sem)

    def gwait(rt, rp, sem):
        pltpu.make_async_copy(tok_t.at[pl.ds(0, CHUNK)], rt, sem).wait()
        pltpu.make_async_copy(pos_t.at[pl.ds(0, CHUNK)], rp, sem).wait()

    def swait(ro, sem):
        pltpu.make_async_copy(ro, out.at[pl.ds(0, CHUNK)], sem).wait()

    def compute(rt, rp, ro, off):
        rowi = lax.iota(jnp.int32, LANES)
        sidv = idx_s[pl.ds(off, LANES)]
        zeros = jnp.zeros((LANES,), jnp.float32)

        def col1(j, carry):
            s, ssq = carry
            cj = lax.broadcast(j, (LANES,))
            t = plsc.load_gather(rt, [rowi, cj])
            p = plsc.load_gather(rp, [rowi, cj])
            g = plsc.load_gather(sv, [sidv, cj])
            x = t + p + g
            xT[j, :] = x
            return (s + x, ssq + x * x)

        s, ssq = lax.fori_loop(0, HID, col1, (zeros, zeros), unroll=8)
        mean = s * (1.0 / HID)
        inv = _rsqrt16(ssq * (1.0 / HID) - mean * mean + EPS)

        def col2(k, carry):
            j0 = k * LANES
            gvv = gv[pl.ds(j0, LANES)]
            bvv = bv[pl.ds(j0, LANES)]
            for i in range(LANES):
                x = xT[j0 + i, :]
                v = (x - mean) * inv * gvv[i] + bvv[i]
                plsc.store_scatter(ro, [rowi, lax.broadcast(j0 + i, (LANES,))], v)
            return carry

        lax.fori_loop(0, VPR, col2, 0)

    gstart(0, rt0, rp0, sg0)
    gstart(1, rt1, rp1, sg1)
    nc2 = nchunk // 2

    def pair(c2, carry):
        e = c2 * 2
        o = e + 1
        gwait(rt0, rp0, sg0)

        @pl.when(c2 > 0)
        def _():
            swait(ro0, ss0)

        compute(rt0, rp0, ro0, e * CHUNK)
        pltpu.async_copy(ro0, out.at[pl.ds(base + e * CHUNK, CHUNK)], ss0)

        @pl.when(c2 + 1 < nc2)
        def _():
            gstart(e + 2, rt0, rp0, sg0)

        gwait(rt1, rp1, sg1)

        @pl.when(c2 > 0)
        def _():
            swait(ro1, ss1)

        compute(rt1, rp1, ro1, o * CHUNK)
        pltpu.async_copy(ro1, out.at[pl.ds(base + o * CHUNK, CHUNK)], ss1)

        @pl.when(c2 + 1 < nc2)
        def _():
            gstart(o + 2, rt1, rp1, sg1)

        return carry

    lax.fori_loop(0, nc2, pair, 0)
    swait(ro0, ss0)
    swait(ro1, ss1)


def kernel(token_ids, segment_ids, position_ids, tok_table, seg_table,
           pos_table, gamma, beta):
    b, s = token_ids.shape
    n = b * s
    tpw = n // NW
    nchunk = tpw // CHUNK
    tid = token_ids.reshape(n).astype(jnp.int32)
    sid = segment_ids.reshape(n).astype(jnp.int32)
    pid = position_ids.reshape(n).astype(jnp.int32)

    body = functools.partial(_emb_body, tpw=tpw, nchunk=nchunk)
    fn = pl.kernel(
        body,
        mesh=plsc.VectorSubcoreMesh(core_axis_name="c", subcore_axis_name="s"),
        out_type=jax.ShapeDtypeStruct((n, HID), jnp.float32),
        compiler_params=pltpu.CompilerParams(use_tc_tiling_on_sc=False, needs_layout_passes=False),
        scratch_types=[
            pltpu.VMEM((tpw,), jnp.int32),
            pltpu.VMEM((tpw + LANES,), jnp.int32),
            pltpu.VMEM((tpw,), jnp.int32),
            pltpu.VMEM((CHUNK, HID), jnp.float32),
            pltpu.VMEM((CHUNK, HID), jnp.float32),
            pltpu.VMEM((CHUNK, HID), jnp.float32),
            pltpu.VMEM((CHUNK, HID), jnp.float32),
            pltpu.VMEM((CHUNK, HID), jnp.float32),
            pltpu.VMEM((CHUNK, HID), jnp.float32),
            pltpu.VMEM((HID, LANES), jnp.float32),
            pltpu.VMEM((2, HID), jnp.float32),
            pltpu.VMEM((HID,), jnp.float32),
            pltpu.VMEM((HID,), jnp.float32),
            pltpu.SemaphoreType.DMA,
            pltpu.SemaphoreType.DMA,
            pltpu.SemaphoreType.DMA,
            pltpu.SemaphoreType.DMA,
        ],
    )
    out = fn(tok_table, seg_table, pos_table, tid, sid, pid, gamma, beta)
    return out.reshape(b, s, HID)


# speedup vs baseline: 6.7805x; 6.7805x over previous
import functools

import jax
import jax.numpy as jnp
from jax import lax
from jax.experimental import pallas as pl
from jax.experimental.pallas import tpu as pltpu
from jax.experimental.pallas import tpu_sc as plsc

HID = 768
LANES = 16
VPR = HID // LANES
NW = 32
CHUNK = 16
EPS = 1e-5
LN_ROWS = 1024


def _emb_body(tok_t, seg_t, pos_t, tid, sid, pid, out,
              idx_t, idx_s, idx_p, rt0, rp0, rt1, rp1, ro0, ro1, sv,
              sg0, sg1, ss0, ss1, tpw, nchunk):
    wid = lax.axis_index("s") * 2 + lax.axis_index("c")
    base = wid * tpw
    pltpu.sync_copy(tid.at[pl.ds(base, tpw)], idx_t)
    pltpu.sync_copy(sid.at[pl.ds(base, tpw)], idx_s.at[pl.ds(0, tpw)])
    pltpu.sync_copy(pid.at[pl.ds(base, tpw)], idx_p)
    pltpu.sync_copy(seg_t, sv)

    def gstart(c, rt, rp, sem):
        off = c * CHUNK
        pltpu.async_copy(tok_t.at[idx_t.at[pl.ds(off, CHUNK)]], rt, sem)
        pltpu.async_copy(pos_t.at[idx_p.at[pl.ds(off, CHUNK)]], rp, sem)

    def gwait(rt, rp, sem):
        pltpu.make_async_copy(tok_t.at[pl.ds(0, CHUNK)], rt, sem).wait()
        pltpu.make_async_copy(pos_t.at[pl.ds(0, CHUNK)], rp, sem).wait()

    def swait(ro, sem):
        pltpu.make_async_copy(ro, out.at[pl.ds(0, CHUNK)], sem).wait()

    def compute(rt, rp, ro, off):
        def token(i, tc):
            sid_ = idx_s[pl.ds(off + i, LANES)][0]
            for j in range(VPR):
                sl = pl.ds(j * LANES, LANES)
                ro[i, sl] = rt[i, sl] + rp[i, sl] + sv[sid_, sl]
            return tc

        lax.fori_loop(0, CHUNK, token, 0)

    gstart(0, rt0, rp0, sg0)
    gstart(1, rt1, rp1, sg1)
    nc2 = nchunk // 2

    def pair(c2, carry):
        e = c2 * 2
        o = e + 1
        gwait(rt0, rp0, sg0)

        @pl.when(c2 > 0)
        def _():
            swait(ro0, ss0)

        compute(rt0, rp0, ro0, e * CHUNK)
        pltpu.async_copy(ro0, out.at[pl.ds(base + e * CHUNK, CHUNK)], ss0)

        @pl.when(c2 + 1 < nc2)
        def _():
            gstart(e + 2, rt0, rp0, sg0)

        gwait(rt1, rp1, sg1)

        @pl.when(c2 > 0)
        def _():
            swait(ro1, ss1)

        compute(rt1, rp1, ro1, o * CHUNK)
        pltpu.async_copy(ro1, out.at[pl.ds(base + o * CHUNK, CHUNK)], ss1)

        @pl.when(c2 + 1 < nc2)
        def _():
            gstart(o + 2, rt1, rp1, sg1)

        return carry

    lax.fori_loop(0, nc2, pair, 0)
    swait(ro0, ss0)
    swait(ro1, ss1)


def _ln_body(x_ref, g_ref, b_ref, o_ref):
    x = x_ref[...]
    mean = jnp.mean(x, axis=-1, keepdims=True)
    xc = x - mean
    var = jnp.mean(xc * xc, axis=-1, keepdims=True)
    o_ref[...] = xc * lax.rsqrt(var + EPS) * g_ref[...] + b_ref[...]


def kernel(token_ids, segment_ids, position_ids, tok_table, seg_table,
           pos_table, gamma, beta):
    b, s = token_ids.shape
    n = b * s
    tpw = n // NW
    nchunk = tpw // CHUNK
    tid = token_ids.reshape(n).astype(jnp.int32)
    sid = segment_ids.reshape(n).astype(jnp.int32)
    pid = position_ids.reshape(n).astype(jnp.int32)

    body = functools.partial(_emb_body, tpw=tpw, nchunk=nchunk)
    fn = pl.kernel(
        body,
        mesh=plsc.VectorSubcoreMesh(core_axis_name="c", subcore_axis_name="s"),
        out_type=jax.ShapeDtypeStruct((n, HID), jnp.float32),
        scratch_types=[
            pltpu.VMEM((tpw,), jnp.int32),
            pltpu.VMEM((tpw + LANES,), jnp.int32),
            pltpu.VMEM((tpw,), jnp.int32),
            pltpu.VMEM((CHUNK, HID), jnp.float32),
            pltpu.VMEM((CHUNK, HID), jnp.float32),
            pltpu.VMEM((CHUNK, HID), jnp.float32),
            pltpu.VMEM((CHUNK, HID), jnp.float32),
            pltpu.VMEM((CHUNK, HID), jnp.float32),
            pltpu.VMEM((CHUNK, HID), jnp.float32),
            pltpu.VMEM((2, HID), jnp.float32),
            pltpu.SemaphoreType.DMA,
            pltpu.SemaphoreType.DMA,
            pltpu.SemaphoreType.DMA,
            pltpu.SemaphoreType.DMA,
        ],
    )
    emb = fn(tok_table, seg_table, pos_table, tid, sid, pid)

    out = pl.pallas_call(
        _ln_body,
        grid=(n // LN_ROWS,),
        in_specs=[
            pl.BlockSpec((LN_ROWS, HID), lambda i: (i, 0)),
            pl.BlockSpec((1, HID), lambda i: (0, 0)),
            pl.BlockSpec((1, HID), lambda i: (0, 0)),
        ],
        out_specs=pl.BlockSpec((LN_ROWS, HID), lambda i: (i, 0)),
        out_shape=jax.ShapeDtypeStruct((n, HID), jnp.float32),
    )(emb, gamma.reshape(1, HID), beta.reshape(1, HID))
    return out.reshape(b, s, HID)


# speedup vs baseline: 8.8912x; 1.3113x over previous
import functools

import jax
import jax.numpy as jnp
from jax import lax
from jax.experimental import pallas as pl
from jax.experimental.pallas import tpu as pltpu
from jax.experimental.pallas import tpu_sc as plsc

HID = 768
LANES = 16
VPR = HID // LANES
NW = 32
CHUNK = 16
EPS = 1e-5
LN_ROWS = 1024


def _emb_body(tok_t, seg_t, pos_t, tid, sid, pid, out,
              idx_t, idx_s, idx_p, rt0, rp0, rt1, rp1, ro0, ro1, sv,
              sg0, sg1, ss0, ss1, tpw, nchunk):
    wid = lax.axis_index("s") * 2 + lax.axis_index("c")
    base = wid * tpw
    pltpu.sync_copy(tid.at[pl.ds(base, tpw)], idx_t)
    pltpu.sync_copy(sid.at[pl.ds(base, tpw)], idx_s.at[pl.ds(0, tpw)])
    pltpu.sync_copy(pid.at[pl.ds(base, tpw)], idx_p)
    pltpu.sync_copy(seg_t, sv)

    def gstart(c, rt, rp, sem):
        off = c * CHUNK
        pltpu.async_copy(tok_t.at[idx_t.at[pl.ds(off, CHUNK)]], rt, sem)
        pltpu.async_copy(pos_t.at[idx_p.at[pl.ds(off, CHUNK)]], rp, sem)

    def gwait(rt, rp, sem):
        pltpu.make_async_copy(tok_t.at[pl.ds(0, CHUNK)], rt, sem).wait()
        pltpu.make_async_copy(pos_t.at[pl.ds(0, CHUNK)], rp, sem).wait()

    def swait(ro, sem):
        pltpu.make_async_copy(ro, out.at[pl.ds(0, CHUNK)], sem).wait()

    def compute(rt, rp, ro, off):
        @plsc.parallel_loop(0, CHUNK, unroll=2)
        def token(i):
            sid_ = idx_s[pl.ds(off + i, LANES)][0]
            for j in range(VPR):
                sl = pl.ds(j * LANES, LANES)
                ro[i, sl] = rt[i, sl] + rp[i, sl] + sv[sid_, sl]

    gstart(0, rt0, rp0, sg0)
    gstart(1, rt1, rp1, sg1)
    nc2 = nchunk // 2

    def pair(c2, carry):
        e = c2 * 2
        o = e + 1
        gwait(rt0, rp0, sg0)

        @pl.when(c2 > 0)
        def _():
            swait(ro0, ss0)

        compute(rt0, rp0, ro0, e * CHUNK)
        pltpu.async_copy(ro0, out.at[pl.ds(base + e * CHUNK, CHUNK)], ss0)

        @pl.when(c2 + 1 < nc2)
        def _():
            gstart(e + 2, rt0, rp0, sg0)

        gwait(rt1, rp1, sg1)

        @pl.when(c2 > 0)
        def _():
            swait(ro1, ss1)

        compute(rt1, rp1, ro1, o * CHUNK)
        pltpu.async_copy(ro1, out.at[pl.ds(base + o * CHUNK, CHUNK)], ss1)

        @pl.when(c2 + 1 < nc2)
        def _():
            gstart(o + 2, rt1, rp1, sg1)

        return carry

    lax.fori_loop(0, nc2, pair, 0)
    swait(ro0, ss0)
    swait(ro1, ss1)


def _ln_body(x_ref, g_ref, b_ref, o_ref):
    x = x_ref[...]
    mean = jnp.mean(x, axis=-1, keepdims=True)
    xc = x - mean
    var = jnp.mean(xc * xc, axis=-1, keepdims=True)
    o_ref[...] = xc * lax.rsqrt(var + EPS) * g_ref[...] + b_ref[...]


def kernel(token_ids, segment_ids, position_ids, tok_table, seg_table,
           pos_table, gamma, beta):
    b, s = token_ids.shape
    n = b * s
    tpw = n // NW
    nchunk = tpw // CHUNK
    tid = token_ids.reshape(n).astype(jnp.int32)
    sid = segment_ids.reshape(n).astype(jnp.int32)
    pid = position_ids.reshape(n).astype(jnp.int32)

    body = functools.partial(_emb_body, tpw=tpw, nchunk=nchunk)
    fn = pl.kernel(
        body,
        mesh=plsc.VectorSubcoreMesh(core_axis_name="c", subcore_axis_name="s"),
        out_type=jax.ShapeDtypeStruct((n, HID), jnp.float32),
        scratch_types=[
            pltpu.VMEM((tpw,), jnp.int32),
            pltpu.VMEM((tpw + LANES,), jnp.int32),
            pltpu.VMEM((tpw,), jnp.int32),
            pltpu.VMEM((CHUNK, HID), jnp.float32),
            pltpu.VMEM((CHUNK, HID), jnp.float32),
            pltpu.VMEM((CHUNK, HID), jnp.float32),
            pltpu.VMEM((CHUNK, HID), jnp.float32),
            pltpu.VMEM((CHUNK, HID), jnp.float32),
            pltpu.VMEM((CHUNK, HID), jnp.float32),
            pltpu.VMEM((2, HID), jnp.float32),
            pltpu.SemaphoreType.DMA,
            pltpu.SemaphoreType.DMA,
            pltpu.SemaphoreType.DMA,
            pltpu.SemaphoreType.DMA,
        ],
    )
    emb = fn(tok_table, seg_table, pos_table, tid, sid, pid)

    out = pl.pallas_call(
        _ln_body,
        grid=(n // LN_ROWS,),
        in_specs=[
            pl.BlockSpec((LN_ROWS, HID), lambda i: (i, 0)),
            pl.BlockSpec((1, HID), lambda i: (0, 0)),
            pl.BlockSpec((1, HID), lambda i: (0, 0)),
        ],
        out_specs=pl.BlockSpec((LN_ROWS, HID), lambda i: (i, 0)),
        out_shape=jax.ShapeDtypeStruct((n, HID), jnp.float32),
    )(emb, gamma.reshape(1, HID), beta.reshape(1, HID))
    return out.reshape(b, s, HID)


# speedup vs baseline: 9.9542x; 1.1196x over previous
import functools

import jax
import jax.numpy as jnp
from jax import lax
from jax.experimental import pallas as pl
from jax.experimental.pallas import tpu as pltpu
from jax.experimental.pallas import tpu_sc as plsc

HID = 768
LANES = 16
VPR = HID // LANES
NW = 32
CHUNK = 16
EPS = 1e-5
LN_ROWS = 1024


def _emb_body(tok_t, seg_t, pos_t, tid, sid, pid, out,
              idx_t, idx_s, idx_p, rt0, rp0, rt1, rp1, ro0, ro1, sv,
              sg0, sg1, ss0, ss1, tpw, nchunk):
    wid = lax.axis_index("s") * 2 + lax.axis_index("c")
    base = wid * tpw
    pltpu.sync_copy(tid.at[pl.ds(base, tpw)], idx_t)
    pltpu.sync_copy(sid.at[pl.ds(base, tpw)], idx_s.at[pl.ds(0, tpw)])
    pltpu.sync_copy(pid.at[pl.ds(base, tpw)], idx_p)
    pltpu.sync_copy(seg_t, sv)

    def gstart(c, rt, rp, sem):
        off = c * CHUNK
        pltpu.async_copy(tok_t.at[idx_t.at[pl.ds(off, CHUNK)]], rt, sem)
        pltpu.async_copy(pos_t.at[idx_p.at[pl.ds(off, CHUNK)]], rp, sem)

    def gwait(rt, rp, sem):
        pltpu.make_async_copy(tok_t.at[pl.ds(0, CHUNK)], rt, sem).wait()
        pltpu.make_async_copy(pos_t.at[pl.ds(0, CHUNK)], rp, sem).wait()

    def swait(ro, sem):
        pltpu.make_async_copy(ro, out.at[pl.ds(0, CHUNK)], sem).wait()

    def compute(rt, rp, ro, off):
        @plsc.parallel_loop(0, CHUNK, unroll=4)
        def token(i):
            sid_ = idx_s[pl.ds(off + i, LANES)][0]
            for j in range(VPR):
                sl = pl.ds(j * LANES, LANES)
                ro[i, sl] = rt[i, sl] + rp[i, sl] + sv[sid_, sl]

    gstart(0, rt0, rp0, sg0)
    gstart(1, rt1, rp1, sg1)
    nc2 = nchunk // 2

    def pair(c2, carry):
        e = c2 * 2
        o = e + 1
        gwait(rt0, rp0, sg0)

        @pl.when(c2 > 0)
        def _():
            swait(ro0, ss0)

        compute(rt0, rp0, ro0, e * CHUNK)
        pltpu.async_copy(ro0, out.at[pl.ds(base + e * CHUNK, CHUNK)], ss0)

        @pl.when(c2 + 1 < nc2)
        def _():
            gstart(e + 2, rt0, rp0, sg0)

        gwait(rt1, rp1, sg1)

        @pl.when(c2 > 0)
        def _():
            swait(ro1, ss1)

        compute(rt1, rp1, ro1, o * CHUNK)
        pltpu.async_copy(ro1, out.at[pl.ds(base + o * CHUNK, CHUNK)], ss1)

        @pl.when(c2 + 1 < nc2)
        def _():
            gstart(o + 2, rt1, rp1, sg1)

        return carry

    lax.fori_loop(0, nc2, pair, 0)
    swait(ro0, ss0)
    swait(ro1, ss1)


def _ln_body(x_ref, g_ref, b_ref, o_ref):
    x = x_ref[...]
    mean = jnp.mean(x, axis=-1, keepdims=True)
    xc = x - mean
    var = jnp.mean(xc * xc, axis=-1, keepdims=True)
    o_ref[...] = xc * lax.rsqrt(var + EPS) * g_ref[...] + b_ref[...]


def kernel(token_ids, segment_ids, position_ids, tok_table, seg_table,
           pos_table, gamma, beta):
    b, s = token_ids.shape
    n = b * s
    tpw = n // NW
    nchunk = tpw // CHUNK
    tid = token_ids.reshape(n).astype(jnp.int32)
    sid = segment_ids.reshape(n).astype(jnp.int32)
    pid = position_ids.reshape(n).astype(jnp.int32)

    body = functools.partial(_emb_body, tpw=tpw, nchunk=nchunk)
    fn = pl.kernel(
        body,
        mesh=plsc.VectorSubcoreMesh(core_axis_name="c", subcore_axis_name="s"),
        out_type=jax.ShapeDtypeStruct((n, HID), jnp.float32),
        scratch_types=[
            pltpu.VMEM((tpw,), jnp.int32),
            pltpu.VMEM((tpw + LANES,), jnp.int32),
            pltpu.VMEM((tpw,), jnp.int32),
            pltpu.VMEM((CHUNK, HID), jnp.float32),
            pltpu.VMEM((CHUNK, HID), jnp.float32),
            pltpu.VMEM((CHUNK, HID), jnp.float32),
            pltpu.VMEM((CHUNK, HID), jnp.float32),
            pltpu.VMEM((CHUNK, HID), jnp.float32),
            pltpu.VMEM((CHUNK, HID), jnp.float32),
            pltpu.VMEM((2, HID), jnp.float32),
            pltpu.SemaphoreType.DMA,
            pltpu.SemaphoreType.DMA,
            pltpu.SemaphoreType.DMA,
            pltpu.SemaphoreType.DMA,
        ],
    )
    emb = fn(tok_table, seg_table, pos_table, tid, sid, pid)

    out = pl.pallas_call(
        _ln_body,
        grid=(n // LN_ROWS,),
        in_specs=[
            pl.BlockSpec((LN_ROWS, HID), lambda i: (i, 0)),
            pl.BlockSpec((1, HID), lambda i: (0, 0)),
            pl.BlockSpec((1, HID), lambda i: (0, 0)),
        ],
        out_specs=pl.BlockSpec((LN_ROWS, HID), lambda i: (i, 0)),
        out_shape=jax.ShapeDtypeStruct((n, HID), jnp.float32),
    )(emb, gamma.reshape(1, HID), beta.reshape(1, HID))
    return out.reshape(b, s, HID)
